# baseline (speedup 1.0000x reference)
import functools

import jax
import jax.numpy as jnp
from jax import lax
from jax.experimental import pallas as pl
from jax.experimental.pallas import tpu as pltpu
from jax.experimental.pallas import tpu_sc as plsc

NUM_GROUPS = 512
GROUP_SIZE = 32
EMBED_DIM = 384
B = 8
N = 8192
BM = B * NUM_GROUPS
ROWS = BM * GROUP_SIZE
DPAD = 16


def _fps_body(xs_ref, ys_ref, zs_ref, idx_ref, cx_ref, cy_ref, cz_ref):
    xs = xs_ref[...]
    ys = ys_ref[...]
    zs = zs_ref[...]
    iota_n = lax.broadcasted_iota(jnp.int32, (B, N), 1)
    iota_m = lax.broadcasted_iota(jnp.int32, (B, NUM_GROUPS), 1)

    idx_ref[...] = jnp.zeros((B, NUM_GROUPS), dtype=jnp.int32)
    cx_ref[...] = jnp.zeros((B, NUM_GROUPS), dtype=jnp.float32)
    cy_ref[...] = jnp.zeros((B, NUM_GROUPS), dtype=jnp.float32)
    cz_ref[...] = jnp.zeros((B, NUM_GROUPS), dtype=jnp.float32)

    def body(i, carry):
        dist, far = carry
        onehot = iota_n == far
        cx = jnp.sum(jnp.where(onehot, xs, 0.0), axis=1, keepdims=True)
        cy = jnp.sum(jnp.where(onehot, ys, 0.0), axis=1, keepdims=True)
        cz = jnp.sum(jnp.where(onehot, zs, 0.0), axis=1, keepdims=True)
        sel_i = (iota_m == i).astype(jnp.int32)
        sel_f = sel_i.astype(jnp.float32)
        idx_ref[...] = idx_ref[...] + far * sel_i
        cx_ref[...] = cx_ref[...] + cx * sel_f
        cy_ref[...] = cy_ref[...] + cy * sel_f
        cz_ref[...] = cz_ref[...] + cz * sel_f
        dx = xs - cx
        dy = ys - cy
        dz = zs - cz
        d = dx * dx + dy * dy + dz * dz
        dist = jnp.minimum(dist, d)
        m = jnp.max(dist, axis=1, keepdims=True)
        far = jnp.min(jnp.where(dist == m, iota_n, N), axis=1, keepdims=True)
        return dist, far

    init = (
        jnp.full((B, N), 1e10, dtype=jnp.float32),
        jnp.min(iota_n * 0, axis=1, keepdims=True),
    )
    lax.fori_loop(0, NUM_GROUPS, body, init)


def _fps(xyz):
    out_shapes = (
        jax.ShapeDtypeStruct((B, NUM_GROUPS), jnp.int32),
        jax.ShapeDtypeStruct((B, NUM_GROUPS), jnp.float32),
        jax.ShapeDtypeStruct((B, NUM_GROUPS), jnp.float32),
        jax.ShapeDtypeStruct((B, NUM_GROUPS), jnp.float32),
    )
    idx, cx, cy, cz = pl.pallas_call(
        _fps_body,
        out_shape=out_shapes,
    )(xyz[:, :, 0], xyz[:, :, 1], xyz[:, :, 2])
    centers = jnp.stack([cx, cy, cz], axis=-1)
    return idx, centers


_CBLK = 128


def _select_body(xs_ref, ys_ref, zs_ref, cx_ref, cy_ref, cz_ref, idx_ref,
                 d2_ref):
    b = pl.program_id(0)
    xs = xs_ref[0]
    ys = ys_ref[0]
    zs = zs_ref[0]
    cx = cx_ref[0]
    cy = cy_ref[0]
    cz = cz_ref[0]
    cn2 = xs * xs + ys * ys + zs * zs
    cm2 = cx * cx + cy * cy + cz * cz
    cmat = jnp.concatenate([cx, cy, cz], axis=1)
    xmat = jnp.concatenate([xs, ys, zs], axis=0)
    prod = jnp.dot(cmat, xmat, preferred_element_type=jnp.float32)
    d2_ref[...] = (cm2 - 2.0 * prod) + cn2
    iota_n = lax.broadcasted_iota(jnp.int32, (_CBLK, N), 1)
    iota_k = lax.broadcasted_iota(jnp.int32, (_CBLK, GROUP_SIZE), 1)
    base = b * N
    idx_ref[...] = jnp.zeros((1, _CBLK, GROUP_SIZE), jnp.int32)

    def step(kk, m):
        dist = d2_ref[...]
        amin = jnp.min(jnp.where(dist == m, iota_n, N), axis=1, keepdims=True)
        sel = (iota_k == kk).astype(jnp.int32)
        idx_ref[...] = idx_ref[...] + ((amin + base) * sel)[None]
        newd = jnp.where(iota_n == amin, 1e30, dist)
        d2_ref[...] = newd
        return jnp.min(newd, axis=1, keepdims=True)

    m0 = jnp.min(d2_ref[...], axis=1, keepdims=True)
    lax.fori_loop(0, GROUP_SIZE, step, m0)


def _select(xyz, centers):
    cT = centers[..., None]
    idx = pl.pallas_call(
        _select_body,
        grid=(B, NUM_GROUPS // _CBLK),
        in_specs=[
            pl.BlockSpec((1, 1, N), lambda b, c: (b, 0, 0)),
            pl.BlockSpec((1, 1, N), lambda b, c: (b, 0, 0)),
            pl.BlockSpec((1, 1, N), lambda b, c: (b, 0, 0)),
            pl.BlockSpec((1, _CBLK, 1), lambda b, c: (b, c, 0)),
            pl.BlockSpec((1, _CBLK, 1), lambda b, c: (b, c, 0)),
            pl.BlockSpec((1, _CBLK, 1), lambda b, c: (b, c, 0)),
        ],
        out_specs=pl.BlockSpec((1, _CBLK, GROUP_SIZE), lambda b, c: (b, c, 0)),
        out_shape=jax.ShapeDtypeStruct((B, NUM_GROUPS, GROUP_SIZE), jnp.int32),
        scratch_shapes=[pltpu.VMEM((_CBLK, N), jnp.float32)],
    )(xyz[:, :, 0].reshape(B, 1, N), xyz[:, :, 1].reshape(B, 1, N),
      xyz[:, :, 2].reshape(B, 1, N),
      cT[:, :, 0], cT[:, :, 1], cT[:, :, 2])
    return idx


_IDX_CHUNK = 128
_N_CHUNKS = ROWS // _IDX_CHUNK


def _gather(table, idx_flat):
    info = plsc.get_sparse_core_info()
    nw = info.num_cores * info.num_subcores
    per_w = _N_CHUNKS // nw
    mesh = plsc.VectorSubcoreMesh(core_axis_name="c", subcore_axis_name="s")

    @functools.partial(
        pl.kernel,
        out_type=jax.ShapeDtypeStruct((ROWS, DPAD), jnp.float32),
        mesh=mesh,
        scratch_types=[
            pltpu.VMEM((_IDX_CHUNK,), jnp.int32),
            pltpu.VMEM((_IDX_CHUNK, DPAD), jnp.float32),
            pltpu.SemaphoreType.DMA,
        ],
        compiler_params=pltpu.CompilerParams(use_tc_tiling_on_sc=False),
    )
    def k(table_hbm, idx_hbm, out_hbm, idx_v, rows_v, sem):
        wid = lax.axis_index("s") * info.num_cores + lax.axis_index("c")
        base = wid * per_w

        def body(j, _):
            row = base + j
            pltpu.sync_copy(idx_hbm.at[row], idx_v)
            pltpu.async_copy(table_hbm.at[idx_v], rows_v, sem).wait()
            pltpu.sync_copy(rows_v, out_hbm.at[pl.ds(row * _IDX_CHUNK, _IDX_CHUNK)])
            return 0

        lax.fori_loop(0, per_w, body, 0)

    return k(table, idx_flat.reshape(_N_CHUNKS, _IDX_CHUNK))


_RBLK = 8192


def _mlp_a_body(g_ref, c_ref, w_ref, b_ref, h_ref, s_ref, q_ref):
    i = pl.program_id(0)

    @pl.when(i == 0)
    def _():
        s_ref[...] = jnp.zeros_like(s_ref)
        q_ref[...] = jnp.zeros_like(q_ref)

    local = g_ref[...] - c_ref[...]
    h = jnp.dot(local, w_ref[...], preferred_element_type=jnp.float32) + b_ref[...]
    h_ref[...] = h
    s_ref[...] = s_ref[...] + jnp.sum(h, axis=0, keepdims=True)
    q_ref[...] = q_ref[...] + jnp.sum(h * h, axis=0, keepdims=True)


def _mlp_mid_body(h_ref, sc_ref, sh_ref, w_ref, b_ref, o_ref, s_ref, q_ref):
    i = pl.program_id(0)

    @pl.when(i == 0)
    def _():
        s_ref[...] = jnp.zeros_like(s_ref)
        q_ref[...] = jnp.zeros_like(q_ref)

    a = jnp.maximum(h_ref[...] * sc_ref[...] + sh_ref[...], 0.0)
    h = jnp.dot(a, w_ref[...], preferred_element_type=jnp.float32) + b_ref[...]
    o_ref[...] = h
    s_ref[...] = s_ref[...] + jnp.sum(h, axis=0, keepdims=True)
    q_ref[...] = q_ref[...] + jnp.sum(h * h, axis=0, keepdims=True)


def _mlp_d_body(h_ref, sc_ref, sh_ref, w_ref, b_ref, pe_ref, t_ref):
    a = jnp.maximum(h_ref[...] * sc_ref[...] + sh_ref[...], 0.0)
    h = jnp.dot(a, w_ref[...], preferred_element_type=jnp.float32) + b_ref[...]
    hg = h.reshape(_RBLK // GROUP_SIZE, GROUP_SIZE, EMBED_DIM)
    t_ref[...] = jnp.max(hg, axis=1) + pe_ref[...]


def _mlp_stage_a(g, c, w, b):
    nblk = ROWS // _RBLK
    cout = w.shape[1]
    return pl.pallas_call(
        _mlp_a_body,
        grid=(nblk,),
        in_specs=[
            pl.BlockSpec((_RBLK, DPAD), lambda i: (i, 0)),
            pl.BlockSpec((_RBLK, DPAD), lambda i: (i, 0)),
            pl.BlockSpec(w.shape, lambda i: (0, 0)),
            pl.BlockSpec((1, cout), lambda i: (0, 0)),
        ],
        out_specs=[
            pl.BlockSpec((_RBLK, cout), lambda i: (i, 0)),
            pl.BlockSpec((1, cout), lambda i: (0, 0)),
            pl.BlockSpec((1, cout), lambda i: (0, 0)),
        ],
        out_shape=[
            jax.ShapeDtypeStruct((ROWS, cout), jnp.float32),
            jax.ShapeDtypeStruct((1, cout), jnp.float32),
            jax.ShapeDtypeStruct((1, cout), jnp.float32),
        ],
    )(g, c, w, b)


def _mlp_stage_mid(h, scale, shift, w, b):
    nblk = ROWS // _RBLK
    cin = h.shape[1]
    cout = w.shape[1]
    return pl.pallas_call(
        _mlp_mid_body,
        grid=(nblk,),
        in_specs=[
            pl.BlockSpec((_RBLK, cin), lambda i: (i, 0)),
            pl.BlockSpec((1, cin), lambda i: (0, 0)),
            pl.BlockSpec((1, cin), lambda i: (0, 0)),
            pl.BlockSpec((cin, cout), lambda i: (0, 0)),
            pl.BlockSpec((1, cout), lambda i: (0, 0)),
        ],
        out_specs=[
            pl.BlockSpec((_RBLK, cout), lambda i: (i, 0)),
            pl.BlockSpec((1, cout), lambda i: (0, 0)),
            pl.BlockSpec((1, cout), lambda i: (0, 0)),
        ],
        out_shape=[
            jax.ShapeDtypeStruct((ROWS, cout), jnp.float32),
            jax.ShapeDtypeStruct((1, cout), jnp.float32),
            jax.ShapeDtypeStruct((1, cout), jnp.float32),
        ],
    )(h, scale, shift, w, b)


def _mlp_stage_d(h, scale, shift, w, b, pe):
    nblk = ROWS // _RBLK
    gblk = _RBLK // GROUP_SIZE
    cin = h.shape[1]
    return pl.pallas_call(
        _mlp_d_body,
        grid=(nblk,),
        in_specs=[
            pl.BlockSpec((_RBLK, cin), lambda i: (i, 0)),
            pl.BlockSpec((1, cin), lambda i: (0, 0)),
            pl.BlockSpec((1, cin), lambda i: (0, 0)),
            pl.BlockSpec((cin, EMBED_DIM), lambda i: (0, 0)),
            pl.BlockSpec((1, EMBED_DIM), lambda i: (0, 0)),
            pl.BlockSpec((gblk, EMBED_DIM), lambda i: (i, 0)),
        ],
        out_specs=pl.BlockSpec((gblk, EMBED_DIM), lambda i: (i, 0)),
        out_shape=jax.ShapeDtypeStruct((BM, EMBED_DIM), jnp.float32),
    )(h, scale, shift, w, b, pe)


def _bn_affine(s, q, g, beta):
    mean = s / ROWS
    var = q / ROWS - mean * mean
    scale = g[None, :] / jnp.sqrt(var + 1e-5)
    shift = beta[None, :] - mean * scale
    return scale, shift


def kernel(points, conv1_w, conv1_b, bn1_g, bn1_b, conv2_w, conv2_b, bn2_g,
           bn2_b, conv3_w, conv3_b, bn3_g, bn3_b, conv4_w, conv4_b, pos_w1,
           pos_b1, pos_w2, pos_b2):
    f32 = jnp.float32
    xyz = points[:, :, :3]
    _, centers = _fps(xyz)

    idx = _select(xyz, centers)

    table = jnp.concatenate(
        [xyz, points, jnp.zeros((B, N, DPAD - 9), f32)], axis=-1
    ).reshape(B * N, DPAD)
    gathered = _gather(table, idx.reshape(-1))

    cpad = jnp.concatenate(
        [centers, jnp.zeros((B, NUM_GROUPS, DPAD - 3), f32)], axis=-1)
    cpad = jnp.broadcast_to(
        cpad[:, :, None, :], (B, NUM_GROUPS, GROUP_SIZE, DPAD)
    ).reshape(ROWS, DPAD)

    w1p = jnp.zeros((DPAD, 64), f32).at[:9, :].set(conv1_w.T)
    h1, s1, q1 = _mlp_stage_a(gathered, cpad, w1p, conv1_b[None, :])
    sc1, sh1 = _bn_affine(s1, q1, bn1_g, bn1_b)

    h2, s2, q2 = _mlp_stage_mid(h1, sc1, sh1, conv2_w.T, conv2_b[None, :])
    sc2, sh2 = _bn_affine(s2, q2, bn2_g, bn2_b)

    h3, s3, q3 = _mlp_stage_mid(h2, sc2, sh2, conv3_w.T, conv3_b[None, :])
    sc3, sh3 = _bn_affine(s3, q3, bn3_g, bn3_b)

    pe = (jax.nn.gelu(centers @ pos_w1.T + pos_b1, approximate=False)
          @ pos_w2.T + pos_b2).reshape(BM, EMBED_DIM)

    tokens = _mlp_stage_d(h3, sc3, sh3, conv4_w.T, conv4_b[None, :], pe)
    return tokens.reshape(B, NUM_GROUPS, EMBED_DIM), centers


# speedup vs baseline: 1.1144x; 1.1144x over previous
import functools

import jax
import jax.numpy as jnp
from jax import lax
from jax.experimental import pallas as pl
from jax.experimental.pallas import tpu as pltpu
from jax.experimental.pallas import tpu_sc as plsc

NUM_GROUPS = 512
GROUP_SIZE = 32
EMBED_DIM = 384
B = 8
N = 8192
BM = B * NUM_GROUPS
ROWS = BM * GROUP_SIZE
DPAD = 16


def _fps_body(xs_ref, ys_ref, zs_ref, idx_ref, cx_ref, cy_ref, cz_ref):
    xs = xs_ref[...]
    ys = ys_ref[...]
    zs = zs_ref[...]
    iota_n = lax.broadcasted_iota(jnp.int32, (B, N), 1)
    iota_m = lax.broadcasted_iota(jnp.int32, (B, NUM_GROUPS), 1)

    idx_ref[...] = jnp.zeros((B, NUM_GROUPS), dtype=jnp.int32)
    cx_ref[...] = jnp.zeros((B, NUM_GROUPS), dtype=jnp.float32)
    cy_ref[...] = jnp.zeros((B, NUM_GROUPS), dtype=jnp.float32)
    cz_ref[...] = jnp.zeros((B, NUM_GROUPS), dtype=jnp.float32)

    def body(i, carry):
        dist, far = carry
        onehot = iota_n == far
        cx = jnp.sum(jnp.where(onehot, xs, 0.0), axis=1, keepdims=True)
        cy = jnp.sum(jnp.where(onehot, ys, 0.0), axis=1, keepdims=True)
        cz = jnp.sum(jnp.where(onehot, zs, 0.0), axis=1, keepdims=True)
        sel_i = (iota_m == i).astype(jnp.int32)
        sel_f = sel_i.astype(jnp.float32)
        idx_ref[...] = idx_ref[...] + far * sel_i
        cx_ref[...] = cx_ref[...] + cx * sel_f
        cy_ref[...] = cy_ref[...] + cy * sel_f
        cz_ref[...] = cz_ref[...] + cz * sel_f
        dx = xs - cx
        dy = ys - cy
        dz = zs - cz
        d = dx * dx + dy * dy + dz * dz
        dist = jnp.minimum(dist, d)
        m = jnp.max(dist, axis=1, keepdims=True)
        far = jnp.min(jnp.where(dist == m, iota_n, N), axis=1, keepdims=True)
        return dist, far

    init = (
        jnp.full((B, N), 1e10, dtype=jnp.float32),
        jnp.min(iota_n * 0, axis=1, keepdims=True),
    )
    lax.fori_loop(0, NUM_GROUPS, body, init)


def _fps(xyz):
    out_shapes = (
        jax.ShapeDtypeStruct((B, NUM_GROUPS), jnp.int32),
        jax.ShapeDtypeStruct((B, NUM_GROUPS), jnp.float32),
        jax.ShapeDtypeStruct((B, NUM_GROUPS), jnp.float32),
        jax.ShapeDtypeStruct((B, NUM_GROUPS), jnp.float32),
    )
    idx, cx, cy, cz = pl.pallas_call(
        _fps_body,
        out_shape=out_shapes,
    )(xyz[:, :, 0], xyz[:, :, 1], xyz[:, :, 2])
    centers = jnp.stack([cx, cy, cz], axis=-1)
    return idx, centers


_CBLK = 256


def _select_body(xs_ref, ys_ref, zs_ref, cx_ref, cy_ref, cz_ref, idx_ref,
                 d2_ref):
    b = pl.program_id(0)
    xs = xs_ref[0]
    ys = ys_ref[0]
    zs = zs_ref[0]
    cx = cx_ref[0]
    cy = cy_ref[0]
    cz = cz_ref[0]
    cn2 = xs * xs + ys * ys + zs * zs
    cm2 = cx * cx + cy * cy + cz * cz
    cmat = jnp.concatenate([cx, cy, cz], axis=1)
    xmat = jnp.concatenate([xs, ys, zs], axis=0)
    prod = jnp.dot(cmat, xmat, preferred_element_type=jnp.float32)
    d2_ref[...] = (cm2 - 2.0 * prod) + cn2
    iota_n = lax.broadcasted_iota(jnp.int32, (_CBLK, N), 1)
    iota_k = lax.broadcasted_iota(jnp.int32, (_CBLK, GROUP_SIZE), 1)
    base = b * N
    idx_ref[...] = jnp.zeros((1, _CBLK, GROUP_SIZE), jnp.int32)

    def step(kk, _):
        dist = d2_ref[...]
        m = jnp.min(dist, axis=1, keepdims=True)
        amin = jnp.min(jnp.where(dist == m, iota_n, N), axis=1, keepdims=True)
        sel = (iota_k == kk).astype(jnp.int32)
        idx_ref[...] = idx_ref[...] + ((amin + base) * sel)[None]
        d2_ref[...] = jnp.where(iota_n == amin, 1e30, dist)
        return 0

    lax.fori_loop(0, GROUP_SIZE, step, 0)


def _select(xyz, centers):
    cT = centers[..., None]
    idx = pl.pallas_call(
        _select_body,
        grid=(B, NUM_GROUPS // _CBLK),
        in_specs=[
            pl.BlockSpec((1, 1, N), lambda b, c: (b, 0, 0)),
            pl.BlockSpec((1, 1, N), lambda b, c: (b, 0, 0)),
            pl.BlockSpec((1, 1, N), lambda b, c: (b, 0, 0)),
            pl.BlockSpec((1, _CBLK, 1), lambda b, c: (b, c, 0)),
            pl.BlockSpec((1, _CBLK, 1), lambda b, c: (b, c, 0)),
            pl.BlockSpec((1, _CBLK, 1), lambda b, c: (b, c, 0)),
        ],
        out_specs=pl.BlockSpec((1, _CBLK, GROUP_SIZE), lambda b, c: (b, c, 0)),
        out_shape=jax.ShapeDtypeStruct((B, NUM_GROUPS, GROUP_SIZE), jnp.int32),
        scratch_shapes=[pltpu.VMEM((_CBLK, N), jnp.float32)],
    )(xyz[:, :, 0].reshape(B, 1, N), xyz[:, :, 1].reshape(B, 1, N),
      xyz[:, :, 2].reshape(B, 1, N),
      cT[:, :, 0], cT[:, :, 1], cT[:, :, 2])
    return idx


_IDX_CHUNK = 128
_N_CHUNKS = ROWS // _IDX_CHUNK


def _gather(table, idx_flat):
    info = plsc.get_sparse_core_info()
    nw = info.num_cores * info.num_subcores
    per_w = _N_CHUNKS // nw
    mesh = plsc.VectorSubcoreMesh(core_axis_name="c", subcore_axis_name="s")

    @functools.partial(
        pl.kernel,
        out_type=jax.ShapeDtypeStruct((ROWS, DPAD), jnp.float32),
        mesh=mesh,
        scratch_types=[
            pltpu.VMEM((_IDX_CHUNK,), jnp.int32),
            pltpu.VMEM((_IDX_CHUNK, DPAD), jnp.float32),
            pltpu.SemaphoreType.DMA,
        ],
        compiler_params=pltpu.CompilerParams(use_tc_tiling_on_sc=False),
    )
    def k(table_hbm, idx_hbm, out_hbm, idx_v, rows_v, sem):
        wid = lax.axis_index("s") * info.num_cores + lax.axis_index("c")
        base = wid * per_w

        def body(j, _):
            row = base + j
            pltpu.sync_copy(idx_hbm.at[row], idx_v)
            pltpu.async_copy(table_hbm.at[idx_v], rows_v, sem).wait()
            pltpu.sync_copy(rows_v, out_hbm.at[pl.ds(row * _IDX_CHUNK, _IDX_CHUNK)])
            return 0

        lax.fori_loop(0, per_w, body, 0)

    return k(table, idx_flat.reshape(_N_CHUNKS, _IDX_CHUNK))


_RBLK = 8192


def _mlp_a_body(g_ref, c_ref, w_ref, b_ref, h_ref, s_ref, q_ref):
    i = pl.program_id(0)

    @pl.when(i == 0)
    def _():
        s_ref[...] = jnp.zeros_like(s_ref)
        q_ref[...] = jnp.zeros_like(q_ref)

    local = g_ref[...] - c_ref[...]
    h = jnp.dot(local, w_ref[...], preferred_element_type=jnp.float32) + b_ref[...]
    h_ref[...] = h
    s_ref[...] = s_ref[...] + jnp.sum(h, axis=0, keepdims=True)
    q_ref[...] = q_ref[...] + jnp.sum(h * h, axis=0, keepdims=True)


def _mlp_mid_body(h_ref, sc_ref, sh_ref, w_ref, b_ref, o_ref, s_ref, q_ref):
    i = pl.program_id(0)

    @pl.when(i == 0)
    def _():
        s_ref[...] = jnp.zeros_like(s_ref)
        q_ref[...] = jnp.zeros_like(q_ref)

    a = jnp.maximum(h_ref[...] * sc_ref[...] + sh_ref[...], 0.0)
    h = jnp.dot(a, w_ref[...], preferred_element_type=jnp.float32) + b_ref[...]
    o_ref[...] = h
    s_ref[...] = s_ref[...] + jnp.sum(h, axis=0, keepdims=True)
    q_ref[...] = q_ref[...] + jnp.sum(h * h, axis=0, keepdims=True)


def _mlp_d_body(h_ref, sc_ref, sh_ref, w_ref, b_ref, pe_ref, t_ref):
    a = jnp.maximum(h_ref[...] * sc_ref[...] + sh_ref[...], 0.0)
    h = jnp.dot(a, w_ref[...], preferred_element_type=jnp.float32) + b_ref[...]
    hg = h.reshape(_RBLK // GROUP_SIZE, GROUP_SIZE, EMBED_DIM)
    t_ref[...] = jnp.max(hg, axis=1) + pe_ref[...]


def _mlp_stage_a(g, c, w, b):
    nblk = ROWS // _RBLK
    cout = w.shape[1]
    return pl.pallas_call(
        _mlp_a_body,
        grid=(nblk,),
        in_specs=[
            pl.BlockSpec((_RBLK, DPAD), lambda i: (i, 0)),
            pl.BlockSpec((_RBLK, DPAD), lambda i: (i, 0)),
            pl.BlockSpec(w.shape, lambda i: (0, 0)),
            pl.BlockSpec((1, cout), lambda i: (0, 0)),
        ],
        out_specs=[
            pl.BlockSpec((_RBLK, cout), lambda i: (i, 0)),
            pl.BlockSpec((1, cout), lambda i: (0, 0)),
            pl.BlockSpec((1, cout), lambda i: (0, 0)),
        ],
        out_shape=[
            jax.ShapeDtypeStruct((ROWS, cout), jnp.float32),
            jax.ShapeDtypeStruct((1, cout), jnp.float32),
            jax.ShapeDtypeStruct((1, cout), jnp.float32),
        ],
    )(g, c, w, b)


def _mlp_stage_mid(h, scale, shift, w, b):
    nblk = ROWS // _RBLK
    cin = h.shape[1]
    cout = w.shape[1]
    return pl.pallas_call(
        _mlp_mid_body,
        grid=(nblk,),
        in_specs=[
            pl.BlockSpec((_RBLK, cin), lambda i: (i, 0)),
            pl.BlockSpec((1, cin), lambda i: (0, 0)),
            pl.BlockSpec((1, cin), lambda i: (0, 0)),
            pl.BlockSpec((cin, cout), lambda i: (0, 0)),
            pl.BlockSpec((1, cout), lambda i: (0, 0)),
        ],
        out_specs=[
            pl.BlockSpec((_RBLK, cout), lambda i: (i, 0)),
            pl.BlockSpec((1, cout), lambda i: (0, 0)),
            pl.BlockSpec((1, cout), lambda i: (0, 0)),
        ],
        out_shape=[
            jax.ShapeDtypeStruct((ROWS, cout), jnp.float32),
            jax.ShapeDtypeStruct((1, cout), jnp.float32),
            jax.ShapeDtypeStruct((1, cout), jnp.float32),
        ],
    )(h, scale, shift, w, b)


def _mlp_stage_d(h, scale, shift, w, b, pe):
    nblk = ROWS // _RBLK
    gblk = _RBLK // GROUP_SIZE
    cin = h.shape[1]
    return pl.pallas_call(
        _mlp_d_body,
        grid=(nblk,),
        in_specs=[
            pl.BlockSpec((_RBLK, cin), lambda i: (i, 0)),
            pl.BlockSpec((1, cin), lambda i: (0, 0)),
            pl.BlockSpec((1, cin), lambda i: (0, 0)),
            pl.BlockSpec((cin, EMBED_DIM), lambda i: (0, 0)),
            pl.BlockSpec((1, EMBED_DIM), lambda i: (0, 0)),
            pl.BlockSpec((gblk, EMBED_DIM), lambda i: (i, 0)),
        ],
        out_specs=pl.BlockSpec((gblk, EMBED_DIM), lambda i: (i, 0)),
        out_shape=jax.ShapeDtypeStruct((BM, EMBED_DIM), jnp.float32),
    )(h, scale, shift, w, b, pe)


def _bn_affine(s, q, g, beta):
    mean = s / ROWS
    var = q / ROWS - mean * mean
    scale = g[None, :] / jnp.sqrt(var + 1e-5)
    shift = beta[None, :] - mean * scale
    return scale, shift


def kernel(points, conv1_w, conv1_b, bn1_g, bn1_b, conv2_w, conv2_b, bn2_g,
           bn2_b, conv3_w, conv3_b, bn3_g, bn3_b, conv4_w, conv4_b, pos_w1,
           pos_b1, pos_w2, pos_b2):
    f32 = jnp.float32
    xyz = points[:, :, :3]
    _, centers = _fps(xyz)

    idx = _select(xyz, centers)

    table = jnp.concatenate(
        [xyz, points, jnp.zeros((B, N, DPAD - 9), f32)], axis=-1
    ).reshape(B * N, DPAD)
    gathered = _gather(table, idx.reshape(-1))

    cpad = jnp.concatenate(
        [centers, jnp.zeros((B, NUM_GROUPS, DPAD - 3), f32)], axis=-1)
    cpad = jnp.broadcast_to(
        cpad[:, :, None, :], (B, NUM_GROUPS, GROUP_SIZE, DPAD)
    ).reshape(ROWS, DPAD)

    w1p = jnp.zeros((DPAD, 64), f32).at[:9, :].set(conv1_w.T)
    h1, s1, q1 = _mlp_stage_a(gathered, cpad, w1p, conv1_b[None, :])
    sc1, sh1 = _bn_affine(s1, q1, bn1_g, bn1_b)

    h2, s2, q2 = _mlp_stage_mid(h1, sc1, sh1, conv2_w.T, conv2_b[None, :])
    sc2, sh2 = _bn_affine(s2, q2, bn2_g, bn2_b)

    h3, s3, q3 = _mlp_stage_mid(h2, sc2, sh2, conv3_w.T, conv3_b[None, :])
    sc3, sh3 = _bn_affine(s3, q3, bn3_g, bn3_b)

    pe = (jax.nn.gelu(centers @ pos_w1.T + pos_b1, approximate=False)
          @ pos_w2.T + pos_b2).reshape(BM, EMBED_DIM)

    tokens = _mlp_stage_d(h3, sc3, sh3, conv4_w.T, conv4_b[None, :], pe)
    return tokens.reshape(B, NUM_GROUPS, EMBED_DIM), centers


# speedup vs baseline: 1.1492x; 1.0313x over previous
import functools

import jax
import jax.numpy as jnp
from jax import lax
from jax.experimental import pallas as pl
from jax.experimental.pallas import tpu as pltpu
from jax.experimental.pallas import tpu_sc as plsc

NUM_GROUPS = 512
GROUP_SIZE = 32
EMBED_DIM = 384
B = 8
N = 8192
BM = B * NUM_GROUPS
ROWS = BM * GROUP_SIZE
DPAD = 16


def _fps_body(xs_ref, ys_ref, zs_ref, idx_ref, cx_ref, cy_ref, cz_ref):
    xs = xs_ref[...]
    ys = ys_ref[...]
    zs = zs_ref[...]
    iota_n = lax.broadcasted_iota(jnp.int32, (B, N), 1)
    iota_m = lax.broadcasted_iota(jnp.int32, (B, NUM_GROUPS), 1)

    idx_ref[...] = jnp.zeros((B, NUM_GROUPS), dtype=jnp.int32)
    cx_ref[...] = jnp.zeros((B, NUM_GROUPS), dtype=jnp.float32)
    cy_ref[...] = jnp.zeros((B, NUM_GROUPS), dtype=jnp.float32)
    cz_ref[...] = jnp.zeros((B, NUM_GROUPS), dtype=jnp.float32)

    def body(i, carry):
        dist, far = carry
        onehot = iota_n == far
        cx = jnp.sum(jnp.where(onehot, xs, 0.0), axis=1, keepdims=True)
        cy = jnp.sum(jnp.where(onehot, ys, 0.0), axis=1, keepdims=True)
        cz = jnp.sum(jnp.where(onehot, zs, 0.0), axis=1, keepdims=True)
        sel_i = (iota_m == i).astype(jnp.int32)
        sel_f = sel_i.astype(jnp.float32)
        idx_ref[...] = idx_ref[...] + far * sel_i
        cx_ref[...] = cx_ref[...] + cx * sel_f
        cy_ref[...] = cy_ref[...] + cy * sel_f
        cz_ref[...] = cz_ref[...] + cz * sel_f
        dx = xs - cx
        dy = ys - cy
        dz = zs - cz
        d = dx * dx + dy * dy + dz * dz
        dist = jnp.minimum(dist, d)
        m = jnp.max(dist, axis=1, keepdims=True)
        far = jnp.min(jnp.where(dist == m, iota_n, N), axis=1, keepdims=True)
        return dist, far

    init = (
        jnp.full((B, N), 1e10, dtype=jnp.float32),
        jnp.min(iota_n * 0, axis=1, keepdims=True),
    )
    lax.fori_loop(0, NUM_GROUPS, body, init)


def _fps(xyz):
    out_shapes = (
        jax.ShapeDtypeStruct((B, NUM_GROUPS), jnp.int32),
        jax.ShapeDtypeStruct((B, NUM_GROUPS), jnp.float32),
        jax.ShapeDtypeStruct((B, NUM_GROUPS), jnp.float32),
        jax.ShapeDtypeStruct((B, NUM_GROUPS), jnp.float32),
    )
    idx, cx, cy, cz = pl.pallas_call(
        _fps_body,
        out_shape=out_shapes,
    )(xyz[:, :, 0], xyz[:, :, 1], xyz[:, :, 2])
    centers = jnp.stack([cx, cy, cz], axis=-1)
    return idx, centers


_CBLK = 512


def _select_body(xs_ref, ys_ref, zs_ref, cx_ref, cy_ref, cz_ref, idx_ref,
                 d2_ref):
    b = pl.program_id(0)
    xs = xs_ref[0]
    ys = ys_ref[0]
    zs = zs_ref[0]
    cx = cx_ref[0]
    cy = cy_ref[0]
    cz = cz_ref[0]
    cn2 = xs * xs + ys * ys + zs * zs
    cm2 = cx * cx + cy * cy + cz * cz
    cmat = jnp.concatenate([cx, cy, cz], axis=1)
    xmat = jnp.concatenate([xs, ys, zs], axis=0)
    prod = jnp.dot(cmat, xmat, preferred_element_type=jnp.float32)
    d2_ref[...] = (cm2 - 2.0 * prod) + cn2
    iota_n = lax.broadcasted_iota(jnp.int32, (_CBLK, N), 1)
    iota_k = lax.broadcasted_iota(jnp.int32, (_CBLK, GROUP_SIZE), 1)
    base = b * N
    idx_ref[...] = jnp.zeros((1, _CBLK, GROUP_SIZE), jnp.int32)

    def step(kk, _):
        dist = d2_ref[...]
        m = jnp.min(dist, axis=1, keepdims=True)
        amin = jnp.min(jnp.where(dist == m, iota_n, N), axis=1, keepdims=True)
        sel = (iota_k == kk).astype(jnp.int32)
        idx_ref[...] = idx_ref[...] + ((amin + base) * sel)[None]
        d2_ref[...] = jnp.where(iota_n == amin, 1e30, dist)
        return 0

    lax.fori_loop(0, GROUP_SIZE, step, 0)


def _select(xyz, centers):
    cT = centers[..., None]
    idx = pl.pallas_call(
        _select_body,
        grid=(B, NUM_GROUPS // _CBLK),
        in_specs=[
            pl.BlockSpec((1, 1, N), lambda b, c: (b, 0, 0)),
            pl.BlockSpec((1, 1, N), lambda b, c: (b, 0, 0)),
            pl.BlockSpec((1, 1, N), lambda b, c: (b, 0, 0)),
            pl.BlockSpec((1, _CBLK, 1), lambda b, c: (b, c, 0)),
            pl.BlockSpec((1, _CBLK, 1), lambda b, c: (b, c, 0)),
            pl.BlockSpec((1, _CBLK, 1), lambda b, c: (b, c, 0)),
        ],
        out_specs=pl.BlockSpec((1, _CBLK, GROUP_SIZE), lambda b, c: (b, c, 0)),
        out_shape=jax.ShapeDtypeStruct((B, NUM_GROUPS, GROUP_SIZE), jnp.int32),
        scratch_shapes=[pltpu.VMEM((_CBLK, N), jnp.float32)],
        compiler_params=pltpu.CompilerParams(
            vmem_limit_bytes=110 * 1024 * 1024),
    )(xyz[:, :, 0].reshape(B, 1, N), xyz[:, :, 1].reshape(B, 1, N),
      xyz[:, :, 2].reshape(B, 1, N),
      cT[:, :, 0], cT[:, :, 1], cT[:, :, 2])
    return idx


_IDX_CHUNK = 128
_N_CHUNKS = ROWS // _IDX_CHUNK


def _gather(table, idx_flat):
    info = plsc.get_sparse_core_info()
    nw = info.num_cores * info.num_subcores
    per_w = _N_CHUNKS // nw
    mesh = plsc.VectorSubcoreMesh(core_axis_name="c", subcore_axis_name="s")

    @functools.partial(
        pl.kernel,
        out_type=jax.ShapeDtypeStruct((ROWS, DPAD), jnp.float32),
        mesh=mesh,
        scratch_types=[
            pltpu.VMEM((_IDX_CHUNK,), jnp.int32),
            pltpu.VMEM((_IDX_CHUNK, DPAD), jnp.float32),
            pltpu.SemaphoreType.DMA,
        ],
        compiler_params=pltpu.CompilerParams(use_tc_tiling_on_sc=False),
    )
    def k(table_hbm, idx_hbm, out_hbm, idx_v, rows_v, sem):
        wid = lax.axis_index("s") * info.num_cores + lax.axis_index("c")
        base = wid * per_w

        def body(j, _):
            row = base + j
            pltpu.sync_copy(idx_hbm.at[row], idx_v)
            pltpu.async_copy(table_hbm.at[idx_v], rows_v, sem).wait()
            pltpu.sync_copy(rows_v, out_hbm.at[pl.ds(row * _IDX_CHUNK, _IDX_CHUNK)])
            return 0

        lax.fori_loop(0, per_w, body, 0)

    return k(table, idx_flat.reshape(_N_CHUNKS, _IDX_CHUNK))


_RBLK = 8192


def _mlp_a_body(g_ref, c_ref, w_ref, b_ref, h_ref, s_ref, q_ref):
    i = pl.program_id(0)

    @pl.when(i == 0)
    def _():
        s_ref[...] = jnp.zeros_like(s_ref)
        q_ref[...] = jnp.zeros_like(q_ref)

    local = g_ref[...] - c_ref[...]
    h = jnp.dot(local, w_ref[...], preferred_element_type=jnp.float32) + b_ref[...]
    h_ref[...] = h
    s_ref[...] = s_ref[...] + jnp.sum(h, axis=0, keepdims=True)
    q_ref[...] = q_ref[...] + jnp.sum(h * h, axis=0, keepdims=True)


def _mlp_mid_body(h_ref, sc_ref, sh_ref, w_ref, b_ref, o_ref, s_ref, q_ref):
    i = pl.program_id(0)

    @pl.when(i == 0)
    def _():
        s_ref[...] = jnp.zeros_like(s_ref)
        q_ref[...] = jnp.zeros_like(q_ref)

    a = jnp.maximum(h_ref[...] * sc_ref[...] + sh_ref[...], 0.0)
    h = jnp.dot(a, w_ref[...], preferred_element_type=jnp.float32) + b_ref[...]
    o_ref[...] = h
    s_ref[...] = s_ref[...] + jnp.sum(h, axis=0, keepdims=True)
    q_ref[...] = q_ref[...] + jnp.sum(h * h, axis=0, keepdims=True)


def _mlp_d_body(h_ref, sc_ref, sh_ref, w_ref, b_ref, pe_ref, t_ref):
    a = jnp.maximum(h_ref[...] * sc_ref[...] + sh_ref[...], 0.0)
    h = jnp.dot(a, w_ref[...], preferred_element_type=jnp.float32) + b_ref[...]
    hg = h.reshape(_RBLK // GROUP_SIZE, GROUP_SIZE, EMBED_DIM)
    t_ref[...] = jnp.max(hg, axis=1) + pe_ref[...]


def _mlp_stage_a(g, c, w, b):
    nblk = ROWS // _RBLK
    cout = w.shape[1]
    return pl.pallas_call(
        _mlp_a_body,
        grid=(nblk,),
        in_specs=[
            pl.BlockSpec((_RBLK, DPAD), lambda i: (i, 0)),
            pl.BlockSpec((_RBLK, DPAD), lambda i: (i, 0)),
            pl.BlockSpec(w.shape, lambda i: (0, 0)),
            pl.BlockSpec((1, cout), lambda i: (0, 0)),
        ],
        out_specs=[
            pl.BlockSpec((_RBLK, cout), lambda i: (i, 0)),
            pl.BlockSpec((1, cout), lambda i: (0, 0)),
            pl.BlockSpec((1, cout), lambda i: (0, 0)),
        ],
        out_shape=[
            jax.ShapeDtypeStruct((ROWS, cout), jnp.float32),
            jax.ShapeDtypeStruct((1, cout), jnp.float32),
            jax.ShapeDtypeStruct((1, cout), jnp.float32),
        ],
    )(g, c, w, b)


def _mlp_stage_mid(h, scale, shift, w, b):
    nblk = ROWS // _RBLK
    cin = h.shape[1]
    cout = w.shape[1]
    return pl.pallas_call(
        _mlp_mid_body,
        grid=(nblk,),
        in_specs=[
            pl.BlockSpec((_RBLK, cin), lambda i: (i, 0)),
            pl.BlockSpec((1, cin), lambda i: (0, 0)),
            pl.BlockSpec((1, cin), lambda i: (0, 0)),
            pl.BlockSpec((cin, cout), lambda i: (0, 0)),
            pl.BlockSpec((1, cout), lambda i: (0, 0)),
        ],
        out_specs=[
            pl.BlockSpec((_RBLK, cout), lambda i: (i, 0)),
            pl.BlockSpec((1, cout), lambda i: (0, 0)),
            pl.BlockSpec((1, cout), lambda i: (0, 0)),
        ],
        out_shape=[
            jax.ShapeDtypeStruct((ROWS, cout), jnp.float32),
            jax.ShapeDtypeStruct((1, cout), jnp.float32),
            jax.ShapeDtypeStruct((1, cout), jnp.float32),
        ],
    )(h, scale, shift, w, b)


def _mlp_stage_d(h, scale, shift, w, b, pe):
    nblk = ROWS // _RBLK
    gblk = _RBLK // GROUP_SIZE
    cin = h.shape[1]
    return pl.pallas_call(
        _mlp_d_body,
        grid=(nblk,),
        in_specs=[
            pl.BlockSpec((_RBLK, cin), lambda i: (i, 0)),
            pl.BlockSpec((1, cin), lambda i: (0, 0)),
            pl.BlockSpec((1, cin), lambda i: (0, 0)),
            pl.BlockSpec((cin, EMBED_DIM), lambda i: (0, 0)),
            pl.BlockSpec((1, EMBED_DIM), lambda i: (0, 0)),
            pl.BlockSpec((gblk, EMBED_DIM), lambda i: (i, 0)),
        ],
        out_specs=pl.BlockSpec((gblk, EMBED_DIM), lambda i: (i, 0)),
        out_shape=jax.ShapeDtypeStruct((BM, EMBED_DIM), jnp.float32),
    )(h, scale, shift, w, b, pe)


def _bn_affine(s, q, g, beta):
    mean = s / ROWS
    var = q / ROWS - mean * mean
    scale = g[None, :] / jnp.sqrt(var + 1e-5)
    shift = beta[None, :] - mean * scale
    return scale, shift


def kernel(points, conv1_w, conv1_b, bn1_g, bn1_b, conv2_w, conv2_b, bn2_g,
           bn2_b, conv3_w, conv3_b, bn3_g, bn3_b, conv4_w, conv4_b, pos_w1,
           pos_b1, pos_w2, pos_b2):
    f32 = jnp.float32
    xyz = points[:, :, :3]
    _, centers = _fps(xyz)

    idx = _select(xyz, centers)

    table = jnp.concatenate(
        [xyz, points, jnp.zeros((B, N, DPAD - 9), f32)], axis=-1
    ).reshape(B * N, DPAD)
    gathered = _gather(table, idx.reshape(-1))

    cpad = jnp.concatenate(
        [centers, jnp.zeros((B, NUM_GROUPS, DPAD - 3), f32)], axis=-1)
    cpad = jnp.broadcast_to(
        cpad[:, :, None, :], (B, NUM_GROUPS, GROUP_SIZE, DPAD)
    ).reshape(ROWS, DPAD)

    w1p = jnp.zeros((DPAD, 64), f32).at[:9, :].set(conv1_w.T)
    h1, s1, q1 = _mlp_stage_a(gathered, cpad, w1p, conv1_b[None, :])
    sc1, sh1 = _bn_affine(s1, q1, bn1_g, bn1_b)

    h2, s2, q2 = _mlp_stage_mid(h1, sc1, sh1, conv2_w.T, conv2_b[None, :])
    sc2, sh2 = _bn_affine(s2, q2, bn2_g, bn2_b)

    h3, s3, q3 = _mlp_stage_mid(h2, sc2, sh2, conv3_w.T, conv3_b[None, :])
    sc3, sh3 = _bn_affine(s3, q3, bn3_g, bn3_b)

    pe = (jax.nn.gelu(centers @ pos_w1.T + pos_b1, approximate=False)
          @ pos_w2.T + pos_b2).reshape(BM, EMBED_DIM)

    tokens = _mlp_stage_d(h3, sc3, sh3, conv4_w.T, conv4_b[None, :], pe)
    return tokens.reshape(B, NUM_GROUPS, EMBED_DIM), centers


# speedup vs baseline: 1.1500x; 1.0007x over previous
import functools

import jax
import jax.numpy as jnp
from jax import lax
from jax.experimental import pallas as pl
from jax.experimental.pallas import tpu as pltpu
from jax.experimental.pallas import tpu_sc as plsc

NUM_GROUPS = 512
GROUP_SIZE = 32
EMBED_DIM = 384
B = 8
N = 8192
BM = B * NUM_GROUPS
ROWS = BM * GROUP_SIZE
DPAD = 16


def _fps_body(xs_ref, ys_ref, zs_ref, idx_ref, cx_ref, cy_ref, cz_ref):
    xs = xs_ref[...]
    ys = ys_ref[...]
    zs = zs_ref[...]
    iota_n = lax.broadcasted_iota(jnp.int32, (B, N), 1)
    iota_m = lax.broadcasted_iota(jnp.int32, (B, NUM_GROUPS), 1)

    idx_ref[...] = jnp.zeros((B, NUM_GROUPS), dtype=jnp.int32)
    cx_ref[...] = jnp.zeros((B, NUM_GROUPS), dtype=jnp.float32)
    cy_ref[...] = jnp.zeros((B, NUM_GROUPS), dtype=jnp.float32)
    cz_ref[...] = jnp.zeros((B, NUM_GROUPS), dtype=jnp.float32)

    def body(i, carry):
        dist, far = carry
        onehot = iota_n == far
        cx = jnp.sum(jnp.where(onehot, xs, 0.0), axis=1, keepdims=True)
        cy = jnp.sum(jnp.where(onehot, ys, 0.0), axis=1, keepdims=True)
        cz = jnp.sum(jnp.where(onehot, zs, 0.0), axis=1, keepdims=True)
        sel_i = (iota_m == i).astype(jnp.int32)
        sel_f = sel_i.astype(jnp.float32)
        idx_ref[...] = idx_ref[...] + far * sel_i
        cx_ref[...] = cx_ref[...] + cx * sel_f
        cy_ref[...] = cy_ref[...] + cy * sel_f
        cz_ref[...] = cz_ref[...] + cz * sel_f
        dx = xs - cx
        dy = ys - cy
        dz = zs - cz
        d = dx * dx + dy * dy + dz * dz
        dist = jnp.minimum(dist, d)
        m = jnp.max(dist, axis=1, keepdims=True)
        far = jnp.min(jnp.where(dist == m, iota_n, N), axis=1, keepdims=True)
        return dist, far

    init = (
        jnp.full((B, N), 1e10, dtype=jnp.float32),
        jnp.min(iota_n * 0, axis=1, keepdims=True),
    )
    lax.fori_loop(0, NUM_GROUPS, body, init)


def _fps(xyz):
    out_shapes = (
        jax.ShapeDtypeStruct((B, NUM_GROUPS), jnp.int32),
        jax.ShapeDtypeStruct((B, NUM_GROUPS), jnp.float32),
        jax.ShapeDtypeStruct((B, NUM_GROUPS), jnp.float32),
        jax.ShapeDtypeStruct((B, NUM_GROUPS), jnp.float32),
    )
    idx, cx, cy, cz = pl.pallas_call(
        _fps_body,
        out_shape=out_shapes,
    )(xyz[:, :, 0], xyz[:, :, 1], xyz[:, :, 2])
    centers = jnp.stack([cx, cy, cz], axis=-1)
    return idx, centers


_CBLK = 512


def _select_body(xs_ref, ys_ref, zs_ref, cx_ref, cy_ref, cz_ref, idx_ref,
                 d2_ref):
    b = pl.program_id(0)
    xs = xs_ref[0]
    ys = ys_ref[0]
    zs = zs_ref[0]
    cx = cx_ref[0]
    cy = cy_ref[0]
    cz = cz_ref[0]
    cn2 = xs * xs + ys * ys + zs * zs
    cm2 = cx * cx + cy * cy + cz * cz
    cmat = jnp.concatenate([cx, cy, cz], axis=1)
    xmat = jnp.concatenate([xs, ys, zs], axis=0)
    prod = jnp.dot(cmat, xmat, preferred_element_type=jnp.float32)
    d2_ref[...] = (cm2 - 2.0 * prod) + cn2
    iota_n = lax.broadcasted_iota(jnp.int32, (_CBLK, N), 1)
    iota_k = lax.broadcasted_iota(jnp.int32, (_CBLK, GROUP_SIZE), 1)
    base = b * N
    idx_ref[...] = jnp.zeros((1, _CBLK, GROUP_SIZE), jnp.int32)

    def step(kk, _):
        dist = d2_ref[...]
        m = jnp.min(dist, axis=1, keepdims=True)
        amin = jnp.min(jnp.where(dist == m, iota_n, N), axis=1, keepdims=True)
        sel = (iota_k == kk).astype(jnp.int32)
        idx_ref[...] = idx_ref[...] + ((amin + base) * sel)[None]
        d2_ref[...] = jnp.where(iota_n == amin, 1e30, dist)
        return 0

    lax.fori_loop(0, GROUP_SIZE, step, 0)


def _select(xyz, centers):
    cT = centers[..., None]
    idx = pl.pallas_call(
        _select_body,
        grid=(B, NUM_GROUPS // _CBLK),
        in_specs=[
            pl.BlockSpec((1, 1, N), lambda b, c: (b, 0, 0)),
            pl.BlockSpec((1, 1, N), lambda b, c: (b, 0, 0)),
            pl.BlockSpec((1, 1, N), lambda b, c: (b, 0, 0)),
            pl.BlockSpec((1, _CBLK, 1), lambda b, c: (b, c, 0)),
            pl.BlockSpec((1, _CBLK, 1), lambda b, c: (b, c, 0)),
            pl.BlockSpec((1, _CBLK, 1), lambda b, c: (b, c, 0)),
        ],
        out_specs=pl.BlockSpec((1, _CBLK, GROUP_SIZE), lambda b, c: (b, c, 0)),
        out_shape=jax.ShapeDtypeStruct((B, NUM_GROUPS, GROUP_SIZE), jnp.int32),
        scratch_shapes=[pltpu.VMEM((_CBLK, N), jnp.float32)],
        compiler_params=pltpu.CompilerParams(
            vmem_limit_bytes=110 * 1024 * 1024,
            dimension_semantics=("parallel", "parallel")),
    )(xyz[:, :, 0].reshape(B, 1, N), xyz[:, :, 1].reshape(B, 1, N),
      xyz[:, :, 2].reshape(B, 1, N),
      cT[:, :, 0], cT[:, :, 1], cT[:, :, 2])
    return idx


_IDX_CHUNK = 128
_N_CHUNKS = ROWS // _IDX_CHUNK


def _gather(table, idx_flat):
    info = plsc.get_sparse_core_info()
    nw = info.num_cores * info.num_subcores
    per_w = _N_CHUNKS // nw
    mesh = plsc.VectorSubcoreMesh(core_axis_name="c", subcore_axis_name="s")

    @functools.partial(
        pl.kernel,
        out_type=jax.ShapeDtypeStruct((ROWS, DPAD), jnp.float32),
        mesh=mesh,
        scratch_types=[
            pltpu.VMEM((_IDX_CHUNK,), jnp.int32),
            pltpu.VMEM((_IDX_CHUNK, DPAD), jnp.float32),
            pltpu.SemaphoreType.DMA,
        ],
        compiler_params=pltpu.CompilerParams(use_tc_tiling_on_sc=False),
    )
    def k(table_hbm, idx_hbm, out_hbm, idx_v, rows_v, sem):
        wid = lax.axis_index("s") * info.num_cores + lax.axis_index("c")
        base = wid * per_w

        def body(j, _):
            row = base + j
            pltpu.sync_copy(idx_hbm.at[row], idx_v)
            pltpu.async_copy(table_hbm.at[idx_v], rows_v, sem).wait()
            pltpu.sync_copy(rows_v, out_hbm.at[pl.ds(row * _IDX_CHUNK, _IDX_CHUNK)])
            return 0

        lax.fori_loop(0, per_w, body, 0)

    return k(table, idx_flat.reshape(_N_CHUNKS, _IDX_CHUNK))


_RBLK = 8192


def _mlp_a_body(g_ref, c_ref, w_ref, b_ref, h_ref, s_ref, q_ref):
    i = pl.program_id(0)

    @pl.when(i == 0)
    def _():
        s_ref[...] = jnp.zeros_like(s_ref)
        q_ref[...] = jnp.zeros_like(q_ref)

    local = g_ref[...] - c_ref[...]
    h = jnp.dot(local, w_ref[...], preferred_element_type=jnp.float32) + b_ref[...]
    h_ref[...] = h
    s_ref[...] = s_ref[...] + jnp.sum(h, axis=0, keepdims=True)
    q_ref[...] = q_ref[...] + jnp.sum(h * h, axis=0, keepdims=True)


def _mlp_mid_body(h_ref, sc_ref, sh_ref, w_ref, b_ref, o_ref, s_ref, q_ref):
    i = pl.program_id(0)

    @pl.when(i == 0)
    def _():
        s_ref[...] = jnp.zeros_like(s_ref)
        q_ref[...] = jnp.zeros_like(q_ref)

    a = jnp.maximum(h_ref[...] * sc_ref[...] + sh_ref[...], 0.0)
    h = jnp.dot(a, w_ref[...], preferred_element_type=jnp.float32) + b_ref[...]
    o_ref[...] = h
    s_ref[...] = s_ref[...] + jnp.sum(h, axis=0, keepdims=True)
    q_ref[...] = q_ref[...] + jnp.sum(h * h, axis=0, keepdims=True)


def _mlp_d_body(h_ref, sc_ref, sh_ref, w_ref, b_ref, pe_ref, t_ref):
    a = jnp.maximum(h_ref[...] * sc_ref[...] + sh_ref[...], 0.0)
    h = jnp.dot(a, w_ref[...], preferred_element_type=jnp.float32) + b_ref[...]
    hg = h.reshape(_RBLK // GROUP_SIZE, GROUP_SIZE, EMBED_DIM)
    t_ref[...] = jnp.max(hg, axis=1) + pe_ref[...]


def _mlp_stage_a(g, c, w, b):
    nblk = ROWS // _RBLK
    cout = w.shape[1]
    return pl.pallas_call(
        _mlp_a_body,
        grid=(nblk,),
        in_specs=[
            pl.BlockSpec((_RBLK, DPAD), lambda i: (i, 0)),
            pl.BlockSpec((_RBLK, DPAD), lambda i: (i, 0)),
            pl.BlockSpec(w.shape, lambda i: (0, 0)),
            pl.BlockSpec((1, cout), lambda i: (0, 0)),
        ],
        out_specs=[
            pl.BlockSpec((_RBLK, cout), lambda i: (i, 0)),
            pl.BlockSpec((1, cout), lambda i: (0, 0)),
            pl.BlockSpec((1, cout), lambda i: (0, 0)),
        ],
        out_shape=[
            jax.ShapeDtypeStruct((ROWS, cout), jnp.float32),
            jax.ShapeDtypeStruct((1, cout), jnp.float32),
            jax.ShapeDtypeStruct((1, cout), jnp.float32),
        ],
    )(g, c, w, b)


def _mlp_stage_mid(h, scale, shift, w, b):
    nblk = ROWS // _RBLK
    cin = h.shape[1]
    cout = w.shape[1]
    return pl.pallas_call(
        _mlp_mid_body,
        grid=(nblk,),
        in_specs=[
            pl.BlockSpec((_RBLK, cin), lambda i: (i, 0)),
            pl.BlockSpec((1, cin), lambda i: (0, 0)),
            pl.BlockSpec((1, cin), lambda i: (0, 0)),
            pl.BlockSpec((cin, cout), lambda i: (0, 0)),
            pl.BlockSpec((1, cout), lambda i: (0, 0)),
        ],
        out_specs=[
            pl.BlockSpec((_RBLK, cout), lambda i: (i, 0)),
            pl.BlockSpec((1, cout), lambda i: (0, 0)),
            pl.BlockSpec((1, cout), lambda i: (0, 0)),
        ],
        out_shape=[
            jax.ShapeDtypeStruct((ROWS, cout), jnp.float32),
            jax.ShapeDtypeStruct((1, cout), jnp.float32),
            jax.ShapeDtypeStruct((1, cout), jnp.float32),
        ],
    )(h, scale, shift, w, b)


def _mlp_stage_d(h, scale, shift, w, b, pe):
    nblk = ROWS // _RBLK
    gblk = _RBLK // GROUP_SIZE
    cin = h.shape[1]
    return pl.pallas_call(
        _mlp_d_body,
        grid=(nblk,),
        in_specs=[
            pl.BlockSpec((_RBLK, cin), lambda i: (i, 0)),
            pl.BlockSpec((1, cin), lambda i: (0, 0)),
            pl.BlockSpec((1, cin), lambda i: (0, 0)),
            pl.BlockSpec((cin, EMBED_DIM), lambda i: (0, 0)),
            pl.BlockSpec((1, EMBED_DIM), lambda i: (0, 0)),
            pl.BlockSpec((gblk, EMBED_DIM), lambda i: (i, 0)),
        ],
        out_specs=pl.BlockSpec((gblk, EMBED_DIM), lambda i: (i, 0)),
        out_shape=jax.ShapeDtypeStruct((BM, EMBED_DIM), jnp.float32),
    )(h, scale, shift, w, b, pe)


def _bn_affine(s, q, g, beta):
    mean = s / ROWS
    var = q / ROWS - mean * mean
    scale = g[None, :] / jnp.sqrt(var + 1e-5)
    shift = beta[None, :] - mean * scale
    return scale, shift


def kernel(points, conv1_w, conv1_b, bn1_g, bn1_b, conv2_w, conv2_b, bn2_g,
           bn2_b, conv3_w, conv3_b, bn3_g, bn3_b, conv4_w, conv4_b, pos_w1,
           pos_b1, pos_w2, pos_b2):
    f32 = jnp.float32
    xyz = points[:, :, :3]
    _, centers = _fps(xyz)

    idx = _select(xyz, centers)

    table = jnp.concatenate(
        [xyz, points, jnp.zeros((B, N, DPAD - 9), f32)], axis=-1
    ).reshape(B * N, DPAD)
    gathered = _gather(table, idx.reshape(-1))

    cpad = jnp.concatenate(
        [centers, jnp.zeros((B, NUM_GROUPS, DPAD - 3), f32)], axis=-1)
    cpad = jnp.broadcast_to(
        cpad[:, :, None, :], (B, NUM_GROUPS, GROUP_SIZE, DPAD)
    ).reshape(ROWS, DPAD)

    w1p = jnp.zeros((DPAD, 64), f32).at[:9, :].set(conv1_w.T)
    h1, s1, q1 = _mlp_stage_a(gathered, cpad, w1p, conv1_b[None, :])
    sc1, sh1 = _bn_affine(s1, q1, bn1_g, bn1_b)

    h2, s2, q2 = _mlp_stage_mid(h1, sc1, sh1, conv2_w.T, conv2_b[None, :])
    sc2, sh2 = _bn_affine(s2, q2, bn2_g, bn2_b)

    h3, s3, q3 = _mlp_stage_mid(h2, sc2, sh2, conv3_w.T, conv3_b[None, :])
    sc3, sh3 = _bn_affine(s3, q3, bn3_g, bn3_b)

    pe = (jax.nn.gelu(centers @ pos_w1.T + pos_b1, approximate=False)
          @ pos_w2.T + pos_b2).reshape(BM, EMBED_DIM)

    tokens = _mlp_stage_d(h3, sc3, sh3, conv4_w.T, conv4_b[None, :], pe)
    return tokens.reshape(B, NUM_GROUPS, EMBED_DIM), centers
